# baseline (speedup 1.0000x reference)
import functools

import jax
import jax.numpy as jnp
from jax import lax
from jax.experimental import pallas as pl
from jax.experimental.pallas import tpu as pltpu
from jax.experimental.pallas import tpu_sc as plsc

_N = 5000
_P = 5120
_B = 128
_NBLK = _P // _B
_TH = 0.75
_OFF = 4000.0

_NC, _NS = 2, 16
_NW = _NC * _NS
_ROWS_PER = _P // _NW
_CHN = 80
_NCH = _ROWS_PER // _CHN


def _prep_body(boxest_ref, scol_ref, srow_ref, clsrow_ref, rank_ref,
               data_ref, racc_ref):
    bt = boxest_ref[...]
    offr = clsrow_ref[...] * _OFF
    bofft = bt + offr
    arear = (jnp.maximum(bofft[2:3, :] - bofft[0:1, :], 0.0)
             * jnp.maximum(bofft[3:4, :] - bofft[1:2, :], 0.0))
    slab = jnp.concatenate(
        [bofft, srow_ref[...], bt, arear,
         jnp.zeros((6, _P), jnp.float32)], axis=0)
    data_ref[...] = jnp.transpose(slab)

    racc_ref[...] = jnp.zeros((1, _P), jnp.int32)
    lane = lax.broadcasted_iota(jnp.int32, (1, _B), 1)
    subl = lax.broadcasted_iota(jnp.int32, (_B, 1), 0)

    def blk(p, carry):
        s0 = pl.multiple_of(p * _B, _B)
        sp = scol_ref[pl.ds(s0, _B), :]
        ip = subl + s0

        def ctile(c0):
            sq = srow_ref[0:1, pl.ds(c0, _B)]
            jq = lane + c0
            return ((sq > sp) | ((sq == sp) & (jq < ip))).astype(jnp.int32)

        acc0 = ctile(s0)

        def qloop(q, acc):
            c0 = pl.multiple_of(q * _B, _B)
            c = ctile(c0)
            colsum = jnp.sum(c, axis=0, keepdims=True)
            racc_ref[0:1, pl.ds(c0, _B)] = (
                racc_ref[0:1, pl.ds(c0, _B)] + (_B - colsum))
            return acc + c

        acc = lax.fori_loop(p + 1, _NBLK, qloop, acc0)
        rank_ref[pl.ds(s0, _B), :] = (
            jnp.sum(acc, axis=1, keepdims=True)
            + jnp.transpose(racc_ref[0:1, pl.ds(s0, _B)]))
        return carry

    lax.fori_loop(0, _NBLK, blk, 0)


def _prep(boxes_t, scol, srow, cls_row):
    return pl.pallas_call(
        _prep_body,
        out_shape=[
            jax.ShapeDtypeStruct((_P, 1), jnp.int32),
            jax.ShapeDtypeStruct((_P, 16), jnp.float32),
        ],
        scratch_shapes=[pltpu.VMEM((1, _P), jnp.int32)],
    )(boxes_t, scol, srow, cls_row)


def _sc_scatter(data, rank2d):
    mesh = plsc.VectorSubcoreMesh(
        core_axis_name="c", subcore_axis_name="s",
        num_cores=_NC, num_subcores=_NS)

    @functools.partial(
        pl.kernel,
        out_type=jax.ShapeDtypeStruct((_P, 16), jnp.float32),
        mesh=mesh,
        scratch_types=[
            pltpu.VMEM((_NCH, _CHN), jnp.int32),
            pltpu.VMEM((_ROWS_PER, 16), jnp.float32),
            pltpu.SemaphoreType.DMA,
        ],
        compiler_params=pltpu.CompilerParams(use_tc_tiling_on_sc=False),
    )
    def k(data_hbm, rank_hbm, out_hbm, idx_v, rows_v, sem):
        wid = lax.axis_index("s") * _NC + lax.axis_index("c")
        base = wid * _ROWS_PER
        pltpu.sync_copy(rank_hbm.at[pl.ds(wid * _NCH, _NCH)], idx_v)
        pltpu.sync_copy(data_hbm.at[pl.ds(base, _ROWS_PER)], rows_v)
        for c in range(_NCH):
            pltpu.async_copy(
                rows_v.at[pl.ds(c * _CHN, _CHN)],
                out_hbm.at[idx_v.at[c]],
                sem,
            ).wait()

    return k(data, rank2d)


def _nms_body(d_ref, out_ref, m_ref, sup_ref, t_ref):
    t_ref[...] = jnp.transpose(d_ref[...])
    sup_ref[...] = jnp.zeros((1, _P), jnp.float32)
    tri = (lax.broadcasted_iota(jnp.int32, (_B, _B), 0)
           < lax.broadcasted_iota(jnp.int32, (_B, _B), 1)).astype(jnp.float32)

    def blk(k, carry):
        s0 = pl.multiple_of(k * _B, _B)
        x1i = d_ref[pl.ds(s0, _B), 0:1]
        y1i = d_ref[pl.ds(s0, _B), 1:2]
        x2i = d_ref[pl.ds(s0, _B), 2:3]
        y2i = d_ref[pl.ds(s0, _B), 3:4]
        ai = d_ref[pl.ds(s0, _B), 9:10]

        def tile_sup(c0):
            x1j = t_ref[0:1, pl.ds(c0, _B)]
            y1j = t_ref[1:2, pl.ds(c0, _B)]
            x2j = t_ref[2:3, pl.ds(c0, _B)]
            y2j = t_ref[3:4, pl.ds(c0, _B)]
            aj = t_ref[9:10, pl.ds(c0, _B)]
            xx1 = jnp.maximum(x1i, x1j)
            yy1 = jnp.maximum(y1i, y1j)
            xx2 = jnp.minimum(x2i, x2j)
            yy2 = jnp.minimum(y2i, y2j)
            inter = jnp.maximum(xx2 - xx1, 0.0) * jnp.maximum(yy2 - yy1, 0.0)
            union = ai + aj - inter
            iou = inter / jnp.maximum(union, 1e-9)
            return (iou > _TH).astype(jnp.float32)

        m = tile_sup(s0) * tri
        m_ref[...] = m
        mtot = jnp.sum(m)
        sub0 = sup_ref[0:1, pl.ds(s0, _B)]

        def jcond(c):
            return c[1]

        def jbody(c):
            sub, _ = c
            cnt = jnp.dot(1.0 - sub, m_ref[...],
                          preferred_element_type=jnp.float32)
            new = jnp.maximum(sub0, (cnt > 0.5).astype(jnp.float32))
            return new, jnp.any(new != sub)

        sub = lax.cond(
            mtot > 0.0,
            lambda: lax.while_loop(jcond, jbody, (sub0, True))[0],
            lambda: sub0)
        sup_ref[0:1, pl.ds(s0, _B)] = sub

        subcol = jnp.transpose(sub)
        x1ia = x1i + subcol * 1e9

        def tile_max(c0):
            x1j = t_ref[0:1, pl.ds(c0, _B)]
            y1j = t_ref[1:2, pl.ds(c0, _B)]
            x2j = t_ref[2:3, pl.ds(c0, _B)]
            y2j = t_ref[3:4, pl.ds(c0, _B)]
            aj = t_ref[9:10, pl.ds(c0, _B)]
            xx1 = jnp.maximum(x1ia, x1j)
            yy1 = jnp.maximum(y1i, y1j)
            xx2 = jnp.minimum(x2i, x2j)
            yy2 = jnp.minimum(y2i, y2j)
            inter = jnp.maximum(xx2 - xx1, 0.0) * jnp.maximum(yy2 - yy1, 0.0)
            union = ai + aj - inter
            iou = inter / jnp.maximum(union, 1e-9)
            return jnp.max(iou, axis=0, keepdims=True)

        def upd(c0, red):
            sup_ref[0:1, pl.ds(c0, _B)] = jnp.maximum(
                sup_ref[0:1, pl.ds(c0, _B)],
                (red > _TH).astype(jnp.float32))

        nt = _NBLK - 1 - k
        odd = nt & 1
        first = s0 + _B

        @pl.when(odd == 1)
        def _():
            upd(pl.multiple_of(first, _B), tile_max(pl.multiple_of(first, _B)))

        pstart = first + odd * _B

        def pair(t, carry2):
            c0 = pl.multiple_of(pstart + 2 * t * _B, _B)
            c1 = pl.multiple_of(c0 + _B, _B)
            red = jnp.concatenate([tile_max(c0), tile_max(c1)], axis=1)
            sup_ref[0:1, pl.ds(c0, 2 * _B)] = jnp.maximum(
                sup_ref[0:1, pl.ds(c0, 2 * _B)],
                (red > _TH).astype(jnp.float32))
            return carry2

        lax.fori_loop(0, nt >> 1, pair, 0)
        return carry

    lax.fori_loop(0, _NBLK, blk, 0)

    keepT = jnp.transpose(1.0 - sup_ref[0:1, :])
    out_ref[:, 0:4] = d_ref[:, 5:9] * keepT
    out_ref[:, 4:5] = d_ref[:, 4:5] * keepT
    out_ref[:, 5:8] = jnp.zeros((_P, 3), jnp.float32)


def _nms(sorted_rows):
    return pl.pallas_call(
        _nms_body,
        out_shape=jax.ShapeDtypeStruct((_P, 8), jnp.float32),
        scratch_shapes=[
            pltpu.VMEM((_B, _B), jnp.float32),
            pltpu.VMEM((1, _P), jnp.float32),
            pltpu.VMEM((16, _P), jnp.float32),
        ],
    )(sorted_rows)


def kernel(boxes, scores, classes):
    boxes = boxes.astype(jnp.float32)
    scores = scores.astype(jnp.float32)
    clsf = classes.astype(jnp.float32)
    pad = _P - boxes.shape[0]
    boxes_t = jnp.pad(boxes, ((0, pad), (0, 0))).T
    scol = jnp.pad(scores, (0, pad), constant_values=-1.0).reshape(_P, 1)
    srow = scol.reshape(1, _P)
    cls_row = jnp.pad(clsf, (0, pad)).reshape(1, _P)

    rank, data = _prep(boxes_t, scol, srow, cls_row)
    rank2d = rank.reshape(_P // _CHN, _CHN)
    sorted_rows = _sc_scatter(data, rank2d)
    outp = _nms(sorted_rows)
    return outp[:_N, :5]


# speedup vs baseline: 1.0457x; 1.0457x over previous
import functools

import jax
import jax.numpy as jnp
from jax import lax
from jax.experimental import pallas as pl
from jax.experimental.pallas import tpu as pltpu
from jax.experimental.pallas import tpu_sc as plsc

_N = 5000
_P = 5120
_B = 128
_NBLK = _P // _B
_TH = 0.75
_OFF = 4000.0

_NC, _NS = 2, 16
_NW = _NC * _NS
_ROWS_PER = _P // _NW
_CHN = 80
_NCH = _ROWS_PER // _CHN


def _prep_body(boxest_ref, scol_ref, srow_ref, clsrow_ref, rank_ref,
               data_ref, racc_ref):
    bt = boxest_ref[...]
    offr = clsrow_ref[...] * _OFF
    bofft = bt + offr
    arear = (jnp.maximum(bofft[2:3, :] - bofft[0:1, :], 0.0)
             * jnp.maximum(bofft[3:4, :] - bofft[1:2, :], 0.0))
    slab = jnp.concatenate(
        [bofft, srow_ref[...], bt, arear,
         jnp.zeros((6, _P), jnp.float32)], axis=0)
    data_ref[...] = jnp.transpose(slab)

    racc_ref[...] = jnp.zeros((1, _P), jnp.int32)
    lane = lax.broadcasted_iota(jnp.int32, (1, _B), 1)
    subl = lax.broadcasted_iota(jnp.int32, (_B, 1), 0)

    def blk(p, carry):
        s0 = pl.multiple_of(p * _B, _B)
        sp = scol_ref[pl.ds(s0, _B), :]
        ip = subl + s0

        def ctile(c0):
            sq = srow_ref[0:1, pl.ds(c0, _B)]
            jq = lane + c0
            return ((sq > sp) | ((sq == sp) & (jq < ip))).astype(jnp.int32)

        acc0 = ctile(s0)

        def qtile(c0, acc):
            c = ctile(c0)
            colsum = jnp.sum(c, axis=0, keepdims=True)
            racc_ref[0:1, pl.ds(c0, _B)] = (
                racc_ref[0:1, pl.ds(c0, _B)] + (_B - colsum))
            return acc + c

        nt = _NBLK - 1 - p
        odd = nt & 1
        first = s0 + _B
        acc0 = lax.cond(
            odd == 1,
            lambda: qtile(pl.multiple_of(first, _B), acc0),
            lambda: acc0)
        pstart = first + odd * _B

        def qpair(t, acc):
            c0 = pl.multiple_of(pstart + 2 * t * _B, _B)
            acc = qtile(c0, acc)
            return qtile(pl.multiple_of(c0 + _B, _B), acc)

        acc = lax.fori_loop(0, nt >> 1, qpair, acc0)
        rank_ref[pl.ds(s0, _B), :] = (
            jnp.sum(acc, axis=1, keepdims=True)
            + jnp.transpose(racc_ref[0:1, pl.ds(s0, _B)]))
        return carry

    lax.fori_loop(0, _NBLK, blk, 0)


def _prep(boxes_t, scol, srow, cls_row):
    return pl.pallas_call(
        _prep_body,
        out_shape=[
            jax.ShapeDtypeStruct((_P, 1), jnp.int32),
            jax.ShapeDtypeStruct((_P, 16), jnp.float32),
        ],
        scratch_shapes=[pltpu.VMEM((1, _P), jnp.int32)],
    )(boxes_t, scol, srow, cls_row)


def _sc_scatter(data, rank2d):
    mesh = plsc.VectorSubcoreMesh(
        core_axis_name="c", subcore_axis_name="s",
        num_cores=_NC, num_subcores=_NS)

    @functools.partial(
        pl.kernel,
        out_type=jax.ShapeDtypeStruct((_P, 16), jnp.float32),
        mesh=mesh,
        scratch_types=[
            pltpu.VMEM((_NCH, _CHN), jnp.int32),
            pltpu.VMEM((_ROWS_PER, 16), jnp.float32),
            pltpu.SemaphoreType.DMA,
        ],
        compiler_params=pltpu.CompilerParams(use_tc_tiling_on_sc=False),
    )
    def k(data_hbm, rank_hbm, out_hbm, idx_v, rows_v, sem):
        wid = lax.axis_index("s") * _NC + lax.axis_index("c")
        base = wid * _ROWS_PER
        pltpu.sync_copy(rank_hbm.at[pl.ds(wid * _NCH, _NCH)], idx_v)
        pltpu.sync_copy(data_hbm.at[pl.ds(base, _ROWS_PER)], rows_v)
        for c in range(_NCH):
            pltpu.async_copy(
                rows_v.at[pl.ds(c * _CHN, _CHN)],
                out_hbm.at[idx_v.at[c]],
                sem,
            ).wait()

    return k(data, rank2d)


def _nms_body(d_ref, out_ref, m_ref, sup_ref, t_ref):
    t_ref[...] = jnp.transpose(d_ref[...])
    sup_ref[...] = jnp.zeros((1, _P), jnp.float32)
    tri = (lax.broadcasted_iota(jnp.int32, (_B, _B), 0)
           < lax.broadcasted_iota(jnp.int32, (_B, _B), 1)).astype(jnp.float32)

    def blk(k, carry):
        s0 = pl.multiple_of(k * _B, _B)
        x1i = d_ref[pl.ds(s0, _B), 0:1]
        y1i = d_ref[pl.ds(s0, _B), 1:2]
        x2i = d_ref[pl.ds(s0, _B), 2:3]
        y2i = d_ref[pl.ds(s0, _B), 3:4]
        ai = d_ref[pl.ds(s0, _B), 9:10]

        def tile_sup(c0):
            x1j = t_ref[0:1, pl.ds(c0, _B)]
            y1j = t_ref[1:2, pl.ds(c0, _B)]
            x2j = t_ref[2:3, pl.ds(c0, _B)]
            y2j = t_ref[3:4, pl.ds(c0, _B)]
            aj = t_ref[9:10, pl.ds(c0, _B)]
            xx1 = jnp.maximum(x1i, x1j)
            yy1 = jnp.maximum(y1i, y1j)
            xx2 = jnp.minimum(x2i, x2j)
            yy2 = jnp.minimum(y2i, y2j)
            inter = jnp.maximum(xx2 - xx1, 0.0) * jnp.maximum(yy2 - yy1, 0.0)
            union = ai + aj - inter
            iou = inter / jnp.maximum(union, 1e-9)
            return (iou > _TH).astype(jnp.float32)

        m = tile_sup(s0) * tri
        m_ref[...] = m
        mtot = jnp.sum(m)
        sub0 = sup_ref[0:1, pl.ds(s0, _B)]

        def jcond(c):
            return c[1]

        def jbody(c):
            sub, _ = c
            cnt = jnp.dot(1.0 - sub, m_ref[...],
                          preferred_element_type=jnp.float32)
            new = jnp.maximum(sub0, (cnt > 0.5).astype(jnp.float32))
            return new, jnp.any(new != sub)

        sub = lax.cond(
            mtot > 0.0,
            lambda: lax.while_loop(jcond, jbody, (sub0, True))[0],
            lambda: sub0)
        sup_ref[0:1, pl.ds(s0, _B)] = sub

        subcol = jnp.transpose(sub)
        x1ia = x1i + subcol * 1e9

        def tile_max(c0):
            x1j = t_ref[0:1, pl.ds(c0, _B)]
            y1j = t_ref[1:2, pl.ds(c0, _B)]
            x2j = t_ref[2:3, pl.ds(c0, _B)]
            y2j = t_ref[3:4, pl.ds(c0, _B)]
            aj = t_ref[9:10, pl.ds(c0, _B)]
            xx1 = jnp.maximum(x1ia, x1j)
            yy1 = jnp.maximum(y1i, y1j)
            xx2 = jnp.minimum(x2i, x2j)
            yy2 = jnp.minimum(y2i, y2j)
            inter = jnp.maximum(xx2 - xx1, 0.0) * jnp.maximum(yy2 - yy1, 0.0)
            union = ai + aj - inter
            iou = inter / jnp.maximum(union, 1e-9)
            return jnp.max(iou, axis=0, keepdims=True)

        def upd(c0, red):
            sup_ref[0:1, pl.ds(c0, _B)] = jnp.maximum(
                sup_ref[0:1, pl.ds(c0, _B)],
                (red > _TH).astype(jnp.float32))

        nt = _NBLK - 1 - k
        odd = nt & 1
        first = s0 + _B

        @pl.when(odd == 1)
        def _():
            upd(pl.multiple_of(first, _B), tile_max(pl.multiple_of(first, _B)))

        pstart = first + odd * _B

        def pair(t, carry2):
            c0 = pl.multiple_of(pstart + 2 * t * _B, _B)
            c1 = pl.multiple_of(c0 + _B, _B)
            red = jnp.concatenate([tile_max(c0), tile_max(c1)], axis=1)
            sup_ref[0:1, pl.ds(c0, 2 * _B)] = jnp.maximum(
                sup_ref[0:1, pl.ds(c0, 2 * _B)],
                (red > _TH).astype(jnp.float32))
            return carry2

        lax.fori_loop(0, nt >> 1, pair, 0)
        return carry

    lax.fori_loop(0, _NBLK, blk, 0)

    keep = 1.0 - sup_ref[0:1, :]
    slab = jnp.concatenate(
        [t_ref[5:9, :] * keep, t_ref[4:5, :] * keep,
         jnp.zeros((3, _P), jnp.float32)], axis=0)
    out_ref[...] = jnp.transpose(slab)


def _nms(sorted_rows):
    return pl.pallas_call(
        _nms_body,
        out_shape=jax.ShapeDtypeStruct((_P, 8), jnp.float32),
        scratch_shapes=[
            pltpu.VMEM((_B, _B), jnp.float32),
            pltpu.VMEM((1, _P), jnp.float32),
            pltpu.VMEM((16, _P), jnp.float32),
        ],
    )(sorted_rows)


def kernel(boxes, scores, classes):
    boxes = boxes.astype(jnp.float32)
    scores = scores.astype(jnp.float32)
    clsf = classes.astype(jnp.float32)
    pad = _P - boxes.shape[0]
    boxes_t = jnp.pad(boxes, ((0, pad), (0, 0))).T
    scol = jnp.pad(scores, (0, pad), constant_values=-1.0).reshape(_P, 1)
    srow = scol.reshape(1, _P)
    cls_row = jnp.pad(clsf, (0, pad)).reshape(1, _P)

    rank, data = _prep(boxes_t, scol, srow, cls_row)
    rank2d = rank.reshape(_P // _CHN, _CHN)
    sorted_rows = _sc_scatter(data, rank2d)
    outp = _nms(sorted_rows)
    return outp[:_N, :5]
